# baseline (speedup 1.0000x reference)
import functools

import jax
import jax.numpy as jnp
from jax import lax
from jax.experimental import pallas as pl
from jax.experimental.pallas import tpu as pltpu
from jax.experimental.pallas import tpu_sc as plsc

_CHUNK = 256
_FIRE = 16


@functools.cache
def _make_gather_kernel(V, D, B):
    info = plsc.get_sparse_core_info()
    NC, NS = info.num_cores, info.num_subcores
    NW = NC * NS
    b_per_w = B // NW
    mesh = plsc.VectorSubcoreMesh(core_axis_name="c", subcore_axis_name="s")

    out_sds = jax.ShapeDtypeStruct((B, D), jnp.float32)

    @functools.partial(
        pl.kernel,
        mesh=mesh,
        out_type=(out_sds, out_sds),
        scratch_types=[
            pltpu.VMEM((b_per_w,), jnp.int32),
            pltpu.VMEM((b_per_w,), jnp.int32),
            pltpu.VMEM_SHARED((NS, _CHUNK, D), jnp.float32),
            pltpu.SemaphoreType.DMA,
        ],
        compiler_params=pltpu.CompilerParams(use_tc_tiling_on_sc=True),
    )
    def k(tab_l, tab_r, idx_l, idx_r, out_l, out_r,
          vidx_l, vidx_r, rows_sp, sem):
        sid = lax.axis_index("s")
        wid = sid * NC + lax.axis_index("c")
        base = wid * b_per_w
        pltpu.sync_copy(idx_l.at[pl.ds(base, b_per_w)], vidx_l)
        pltpu.sync_copy(idx_r.at[pl.ds(base, b_per_w)], vidx_r)

        for tab, vidx, out in ((tab_l, vidx_l, out_l), (tab_r, vidx_r, out_r)):
            for c in range(b_per_w // _CHUNK):
                c0 = c * _CHUNK

                def body(i, carry):
                    r0 = i * _FIRE
                    v = vidx[pl.ds(c0 + r0, _FIRE)]
                    for j in range(_FIRE):
                        s = v[j]
                        pltpu.make_async_copy(
                            tab.at[pl.ds(s, 1)],
                            rows_sp.at[sid, pl.ds(r0 + j, 1)],
                            sem,
                        ).start()
                    return carry
                lax.fori_loop(0, _CHUNK // _FIRE, body, 0)

                def drain(i, carry):
                    pltpu.make_async_copy(
                        tab.at[pl.ds(0, 1)],
                        rows_sp.at[sid, pl.ds(0, 1)],
                        sem,
                    ).wait()
                    return carry
                lax.fori_loop(0, _CHUNK, drain, 0)
                pltpu.sync_copy(
                    rows_sp.at[sid],
                    out.at[pl.ds(base + c0, _CHUNK)],
                )

    return k


def kernel(table_left, table_right, indices_left, indices_right):
    V, D = table_left.shape
    (B,) = indices_left.shape
    k = _make_gather_kernel(V, D, B)
    return k(
        table_left,
        table_right,
        indices_left.astype(jnp.int32),
        indices_right.astype(jnp.int32),
    )


# speedup vs baseline: 1.0819x; 1.0819x over previous
import functools

import jax
import jax.numpy as jnp
from jax import lax
from jax.experimental import pallas as pl
from jax.experimental.pallas import tpu as pltpu
from jax.experimental.pallas import tpu_sc as plsc

_CHUNK = 256
_FIRE = 16
_NSEM = 4


@functools.cache
def _make_gather_kernel(V, D, B):
    info = plsc.get_sparse_core_info()
    NC, NS = info.num_cores, info.num_subcores
    NW = NC * NS
    assert B % NW == 0
    b_per_w = B // NW
    n_chunks = b_per_w // _CHUNK
    assert b_per_w % _CHUNK == 0 and _CHUNK % (_FIRE * _NSEM) == 0
    mesh = plsc.VectorSubcoreMesh(core_axis_name="c", subcore_axis_name="s")

    out_sds = jax.ShapeDtypeStruct((B, D), jnp.float32)

    @functools.partial(
        pl.kernel,
        mesh=mesh,
        out_type=(out_sds, out_sds),
        scratch_types=[
            pltpu.VMEM((b_per_w,), jnp.int32),
            pltpu.VMEM((b_per_w,), jnp.int32),
            pltpu.VMEM((_CHUNK, D), jnp.float32),
            pltpu.VMEM((_CHUNK, D), jnp.float32),
            [pltpu.SemaphoreType.DMA] * _NSEM,
            [pltpu.SemaphoreType.DMA] * _NSEM,
        ],
        compiler_params=pltpu.CompilerParams(use_tc_tiling_on_sc=True),
    )
    def k(tab_l, tab_r, idx_l, idx_r, out_l, out_r,
          vidx_l, vidx_r, rows_a, rows_b, sems_a, sems_b):
        wid = lax.axis_index("s") * NC + lax.axis_index("c")
        base = wid * b_per_w
        pltpu.sync_copy(idx_l.at[pl.ds(base, b_per_w)], vidx_l)
        pltpu.sync_copy(idx_r.at[pl.ds(base, b_per_w)], vidx_r)

        def fire_chunk(tab, vidx, rows_v, sems, c0):
            def body(i, carry):
                r0 = i * _FIRE
                v = vidx[pl.ds(c0 + r0, _FIRE)]
                for j in range(_FIRE):
                    s = v[j]
                    pltpu.make_async_copy(
                        tab.at[pl.ds(s, 1)],
                        rows_v.at[pl.ds(r0 + j, 1)],
                        sems[j % _NSEM],
                    ).start()
                return carry
            lax.fori_loop(0, _CHUNK // _FIRE, body, 0)

        def drain_chunk(tab, rows_v, sems):
            def body(i, carry):
                for q in range(_NSEM):
                    pltpu.make_async_copy(
                        tab.at[pl.ds(0, 1)], rows_v.at[pl.ds(0, 1)], sems[q]
                    ).wait()
                return carry
            lax.fori_loop(0, _CHUNK // _NSEM, body, 0)

        for tab, vidx, out in ((tab_l, vidx_l, out_l), (tab_r, vidx_r, out_r)):
            for c in range(n_chunks):
                rows_v, sems = (rows_a, sems_a) if c % 2 == 0 else (rows_b, sems_b)
                fire_chunk(tab, vidx, rows_v, sems, c * _CHUNK)
                drain_chunk(tab, rows_v, sems)
                pltpu.sync_copy(
                    rows_v, out.at[pl.ds(base + c * _CHUNK, _CHUNK)]
                )

    return k


def kernel(table_left, table_right, indices_left, indices_right):
    V, D = table_left.shape
    (B,) = indices_left.shape
    k = _make_gather_kernel(V, D, B)
    return k(
        table_left,
        table_right,
        indices_left.astype(jnp.int32),
        indices_right.astype(jnp.int32),
    )


# speedup vs baseline: 1.0860x; 1.0037x over previous
import functools

import jax
import jax.numpy as jnp
from jax import lax
from jax.experimental import pallas as pl
from jax.experimental.pallas import tpu as pltpu
from jax.experimental.pallas import tpu_sc as plsc

_CHUNK = 256
_FIRE = 16


@functools.cache
def _make_gather_kernel(V, D, B):
    info = plsc.get_sparse_core_info()
    NC, NS = info.num_cores, info.num_subcores
    NW = NC * NS
    assert B % NW == 0
    b_per_w = B // NW
    n_chunks = b_per_w // _CHUNK
    assert b_per_w % _CHUNK == 0 and _CHUNK % _FIRE == 0
    mesh = plsc.VectorSubcoreMesh(core_axis_name="c", subcore_axis_name="s")

    out_sds = jax.ShapeDtypeStruct((B, D), jnp.float32)

    @functools.partial(
        pl.kernel,
        mesh=mesh,
        out_type=(out_sds, out_sds),
        scratch_types=[
            pltpu.VMEM((b_per_w,), jnp.int32),
            pltpu.VMEM((b_per_w,), jnp.int32),
            pltpu.VMEM((_CHUNK, D), jnp.float32),
            pltpu.VMEM((_CHUNK, D), jnp.float32),
            pltpu.SemaphoreType.DMA,
            pltpu.SemaphoreType.DMA,
        ],
        compiler_params=pltpu.CompilerParams(use_tc_tiling_on_sc=True),
    )
    def k(tab_l, tab_r, idx_l, idx_r, out_l, out_r,
          vidx_l, vidx_r, rows_a, rows_b, sem_a, sem_b):
        wid = lax.axis_index("s") * NC + lax.axis_index("c")
        base = wid * b_per_w
        pltpu.sync_copy(idx_l.at[pl.ds(base, b_per_w)], vidx_l)
        pltpu.sync_copy(idx_r.at[pl.ds(base, b_per_w)], vidx_r)

        def fire_chunk(tab, vidx, rows_v, sem, c0):
            def body(i, carry):
                r0 = i * _FIRE
                v = vidx[pl.ds(c0 + r0, _FIRE)]
                for j in range(_FIRE):
                    pltpu.make_async_copy(
                        tab.at[pl.ds(v[j], 1)],
                        rows_v.at[pl.ds(r0 + j, 1)],
                        sem,
                    ).start()
                return carry
            lax.fori_loop(0, _CHUNK // _FIRE, body, 0)

        for tab, vidx, out in ((tab_l, vidx_l, out_l), (tab_r, vidx_r, out_r)):
            for c in range(n_chunks):
                rows_v, sem = (rows_a, sem_a) if c % 2 == 0 else (rows_b, sem_b)
                fire_chunk(tab, vidx, rows_v, sem, c * _CHUNK)
                pltpu.make_async_copy(
                    tab.at[pl.ds(0, _CHUNK)], rows_v, sem
                ).wait()
                pltpu.sync_copy(
                    rows_v, out.at[pl.ds(base + c * _CHUNK, _CHUNK)]
                )

    return k


def kernel(table_left, table_right, indices_left, indices_right):
    V, D = table_left.shape
    (B,) = indices_left.shape
    k = _make_gather_kernel(V, D, B)
    return k(
        table_left,
        table_right,
        indices_left.astype(jnp.int32),
        indices_right.astype(jnp.int32),
    )
